# baseline (speedup 1.0000x reference)
import dataclasses
import functools

import jax
import jax.numpy as jnp
from jax import lax
from jax.experimental import pallas as pl
from jax.experimental.pallas import tpu as pltpu
from jax.experimental.pallas import tpu_sc as plsc

N = 10000
E = 320000
D = 128
NC = 2
NS = 16
NW = NC * NS
LANES = 16
BLK = 80
EPT = E // NW
NBLK = EPT // BLK
NPAD = 10240
RPS = NPAD // NS

_mesh = plsc.VectorSubcoreMesh(
    core_axis_name="c", subcore_axis_name="s", num_cores=NC, num_subcores=NS
)


def _fill(buf, rows, value):
    cols = buf.shape[1] // LANES

    @pl.loop(0, rows)
    def _(i):
        @pl.loop(0, cols)
        def _(j):
            buf[i, pl.ds(j * LANES, LANES)] = jnp.full((LANES,), value, jnp.float32)


@functools.partial(
    pl.kernel,
    out_type=jax.ShapeDtypeStruct((NC, NPAD), jnp.float32),
    mesh=_mesh,
    scratch_types=[
        pltpu.VMEM((NBLK, BLK), jnp.int32),
        pltpu.VMEM((NPAD,), jnp.float32),
        pltpu.VMEM((NS, RPS), jnp.float32),
        pltpu.VMEM_SHARED((NS, NPAD), jnp.float32),
    ],
    compiler_params=dataclasses.replace(pltpu.CompilerParams(),
                                        needs_layout_passes=False),
)
def _deg_sc(dstr_hbm, out_hbm, dstv, hist, red, shared):
    c = lax.axis_index("c")
    s = lax.axis_index("s")
    wid = c * NS + s

    zeros = jnp.zeros((LANES,), jnp.float32)

    @pl.loop(0, NPAD // LANES)
    def _(i):
        hist[pl.ds(i * LANES, LANES)] = zeros

    pltpu.sync_copy(dstr_hbm.at[wid], dstv)
    ones = jnp.ones((LANES,), jnp.float32)

    @pl.loop(0, NBLK)
    def _(j):
        @pl.loop(0, BLK // LANES)
        def _(i):
            plsc.addupdate_scatter(hist, [dstv[j, pl.ds(i * LANES, LANES)]], ones)

    pltpu.sync_copy(hist, shared.at[s])
    plsc.subcore_barrier()

    for k in range(NS):
        pltpu.sync_copy(shared.at[k, pl.ds(s * RPS, RPS)], red.at[k])

    @pl.loop(0, RPS // LANES)
    def _(v):
        acc = red[0, pl.ds(v * LANES, LANES)]
        for k in range(1, NS):
            acc = acc + red[k, pl.ds(v * LANES, LANES)]
        hist[pl.ds(v * LANES, LANES)] = acc

    pltpu.sync_copy(hist.at[pl.ds(0, RPS)], out_hbm.at[c, pl.ds(s * RPS, RPS)])


@functools.partial(
    pl.kernel,
    out_type=jax.ShapeDtypeStruct((NC, NPAD, D), jnp.float32),
    mesh=_mesh,
    scratch_types=[
        pltpu.VMEM((EPT,), jnp.int32),
        pltpu.VMEM((NBLK, BLK), jnp.int32),
        pltpu.VMEM((BLK, D), jnp.float32),
        pltpu.VMEM((BLK, D), jnp.float32),
        pltpu.SemaphoreType.DMA,
        pltpu.SemaphoreType.DMA,
        pltpu.SemaphoreType.DMA,
        pltpu.SemaphoreType.DMA,
        pltpu.VMEM_SHARED((NPAD, D), jnp.float32),
    ],
)
def _hop_sc(g_hbm, srcr_hbm, dstr_hbm, out_hbm, srcv, dstv, rows0, rows1, sem0,
            sem1, sem2, sem3, accum):
    c = lax.axis_index("c")
    s = lax.axis_index("s")
    wid = c * NS + s

    _fill(rows0, BLK, 0.0)

    @pl.loop(0, RPS // BLK)
    def _(i):
        pltpu.sync_copy(rows0, accum.at[pl.ds(s * RPS + i * BLK, BLK)])
    plsc.subcore_barrier()

    pltpu.sync_copy(srcr_hbm.at[wid], srcv)
    pltpu.sync_copy(dstr_hbm.at[wid], dstv)

    pltpu.sync_copy(g_hbm.at[srcv.at[pl.ds(0, BLK)]], rows0)
    pltpu.sync_copy(rows0, accum.at[dstv.at[0]], add=True)

    pltpu.async_copy(g_hbm.at[srcv.at[pl.ds(BLK, BLK)]], rows0, sem0)
    pltpu.async_copy(g_hbm.at[srcv.at[pl.ds(2 * BLK, BLK)]], rows1, sem1)

    @pl.loop(1, NBLK - 2, step=2)
    def _(j):
        pltpu.make_async_copy(g_hbm.at[srcv.at[pl.ds(j * BLK, BLK)]], rows0, sem0).wait()
        pltpu.async_copy(rows0, accum.at[dstv.at[j]], sem2, add=True)
        pltpu.make_async_copy(g_hbm.at[srcv.at[pl.ds((j + 1) * BLK, BLK)]], rows1, sem1).wait()
        pltpu.async_copy(rows1, accum.at[dstv.at[j + 1]], sem3, add=True)
        pltpu.make_async_copy(rows0, accum.at[dstv.at[j]], sem2).wait()
        pltpu.async_copy(g_hbm.at[srcv.at[pl.ds((j + 2) * BLK, BLK)]], rows0, sem0)
        pltpu.make_async_copy(rows1, accum.at[dstv.at[j + 1]], sem3).wait()
        pltpu.async_copy(g_hbm.at[srcv.at[pl.ds((j + 3) * BLK, BLK)]], rows1, sem1)

    pltpu.make_async_copy(g_hbm.at[srcv.at[pl.ds((NBLK - 2) * BLK, BLK)]], rows0, sem0).wait()
    pltpu.sync_copy(rows0, accum.at[dstv.at[NBLK - 2]], add=True)
    pltpu.make_async_copy(g_hbm.at[srcv.at[pl.ds((NBLK - 1) * BLK, BLK)]], rows1, sem1).wait()
    pltpu.sync_copy(rows1, accum.at[dstv.at[NBLK - 1]], add=True)
    plsc.subcore_barrier()

    pltpu.sync_copy(accum.at[pl.ds(s * RPS, RPS)], out_hbm.at[c, pl.ds(s * RPS, RPS)])


_RB = 1280
_GRID = NPAD // _RB

_deg_spec = pl.BlockSpec((NC, _RB), lambda i: (0, i))
_row_spec = pl.BlockSpec((_RB, D), lambda i: (i, 0))
_par_spec = pl.BlockSpec((NC, _RB, D), lambda i: (0, i, 0))


def _norm_of(deg_ref):
    d = (deg_ref[0] + deg_ref[1])[:, None]
    return lax.rsqrt(jnp.maximum(d, 1.0))


def _scale_body(deg_ref, feat_ref, o_ref):
    o_ref[...] = feat_ref[...] * _norm_of(deg_ref)


_scale_call = pl.pallas_call(
    _scale_body,
    grid=(_GRID,),
    in_specs=[_deg_spec, _row_spec],
    out_specs=_row_spec,
    out_shape=jax.ShapeDtypeStruct((NPAD, D), jnp.float32),
)


def _comb_body(deg_ref, p_ref, o_ref):
    d = (deg_ref[0] + deg_ref[1])[:, None]
    o_ref[...] = (p_ref[0] + p_ref[1]) / jnp.maximum(d, 1.0)


_comb_call = pl.pallas_call(
    _comb_body,
    grid=(_GRID,),
    in_specs=[_deg_spec, _par_spec],
    out_specs=_row_spec,
    out_shape=jax.ShapeDtypeStruct((NPAD, D), jnp.float32),
)


def _final_body(deg_ref, p_ref, w_ref, b_ref, o_ref):
    h = (p_ref[0] + p_ref[1]) * _norm_of(deg_ref)
    o_ref[...] = (
        lax.dot_general(h, w_ref[...], (((1,), (1,)), ((), ())),
                        preferred_element_type=jnp.float32)
        + b_ref[...]
    )


_final_call = pl.pallas_call(
    _final_body,
    grid=(_GRID,),
    in_specs=[
        _deg_spec,
        _par_spec,
        pl.BlockSpec((D, D), lambda i: (0, 0)),
        pl.BlockSpec((1, D), lambda i: (0, 0)),
    ],
    out_specs=_row_spec,
    out_shape=jax.ShapeDtypeStruct((NPAD, D), jnp.float32),
)


def kernel(feat, edge_index, W1, b1):
    srcp = edge_index[0].reshape(NW, EPT)
    dstp = edge_index[1].reshape(NW, NBLK, BLK)
    featp = jnp.concatenate([feat, jnp.zeros((NPAD - N, D), feat.dtype)])

    degp = _deg_sc(dstp)
    g1 = _scale_call(degp, featp)
    p1 = _hop_sc(g1, srcp, dstp)
    g2 = _comb_call(degp, p1)
    p2 = _hop_sc(g2, srcp, dstp)
    x = _final_call(degp, p2, W1, b1.reshape(1, D))
    return x[:N]


# speedup vs baseline: 1.0393x; 1.0393x over previous
import dataclasses
import functools

import jax
import jax.numpy as jnp
from jax import lax
from jax.experimental import pallas as pl
from jax.experimental.pallas import tpu as pltpu
from jax.experimental.pallas import tpu_sc as plsc

N = 10000
E = 320000
D = 128
NC = 2
NS = 16
NW = NC * NS
LANES = 16
BLK = 128
EPT = 10240
NBLK = EPT // BLK
NPAD = 10240
RPS = NPAD // NS
ICH = 40

_mesh = plsc.VectorSubcoreMesh(
    core_axis_name="c", subcore_axis_name="s", num_cores=NC, num_subcores=NS
)


def _fill(buf, rows, value):
    cols = buf.shape[1] // LANES

    @pl.loop(0, rows)
    def _(i):
        @pl.loop(0, cols)
        def _(j):
            buf[i, pl.ds(j * LANES, LANES)] = jnp.full((LANES,), value, jnp.float32)


@functools.partial(
    pl.kernel,
    out_type=jax.ShapeDtypeStruct((NC, NPAD), jnp.float32),
    mesh=_mesh,
    scratch_types=[
        pltpu.VMEM((NBLK, BLK), jnp.int32),
        pltpu.VMEM((NPAD,), jnp.float32),
        pltpu.VMEM((NS, RPS), jnp.float32),
        pltpu.VMEM_SHARED((NS, NPAD), jnp.float32),
    ],
    compiler_params=dataclasses.replace(pltpu.CompilerParams(),
                                        needs_layout_passes=False),
)
def _deg_sc(dstr_hbm, out_hbm, dstv, hist, red, shared):
    c = lax.axis_index("c")
    s = lax.axis_index("s")
    wid = c * NS + s

    zeros = jnp.zeros((LANES,), jnp.float32)

    @pl.loop(0, NPAD // LANES)
    def _(i):
        hist[pl.ds(i * LANES, LANES)] = zeros

    pltpu.sync_copy(dstr_hbm.at[wid], dstv)
    ones = jnp.ones((LANES,), jnp.float32)

    @pl.loop(0, NBLK)
    def _(j):
        @pl.loop(0, BLK // LANES)
        def _(i):
            plsc.addupdate_scatter(hist, [dstv[j, pl.ds(i * LANES, LANES)]], ones)

    pltpu.sync_copy(hist, shared.at[s])
    plsc.subcore_barrier()

    for k in range(NS):
        pltpu.sync_copy(shared.at[k, pl.ds(s * RPS, RPS)], red.at[k])

    @pl.loop(0, RPS // LANES)
    def _(v):
        acc = red[0, pl.ds(v * LANES, LANES)]
        for k in range(1, NS):
            acc = acc + red[k, pl.ds(v * LANES, LANES)]
        hist[pl.ds(v * LANES, LANES)] = acc

    pltpu.sync_copy(hist.at[pl.ds(0, RPS)], out_hbm.at[c, pl.ds(s * RPS, RPS)])


@functools.partial(
    pl.kernel,
    out_type=jax.ShapeDtypeStruct((NC, NPAD, D), jnp.float32),
    mesh=_mesh,
    scratch_types=[
        pltpu.VMEM((ICH, BLK), jnp.int32),
        pltpu.VMEM((ICH, BLK), jnp.int32),
        pltpu.VMEM((BLK, D), jnp.float32),
        pltpu.VMEM((BLK, D), jnp.float32),
        pltpu.SemaphoreType.DMA,
        pltpu.SemaphoreType.DMA,
        pltpu.SemaphoreType.DMA,
        pltpu.SemaphoreType.DMA,
        pltpu.VMEM_SHARED((NPAD, D), jnp.float32),
    ],
)
def _hop_sc(g_hbm, srcr_hbm, dstr_hbm, out_hbm, srcv, dstv, rows0, rows1, sem0,
            sem1, sem2, sem3, accum):
    c = lax.axis_index("c")
    s = lax.axis_index("s")
    wid = c * NS + s

    _fill(rows0, BLK, 0.0)

    @pl.loop(0, RPS // BLK)
    def _(i):
        pltpu.sync_copy(rows0, accum.at[pl.ds(s * RPS + i * BLK, BLK)])
    plsc.subcore_barrier()

    @pl.loop(0, NBLK // ICH)
    def _(k):
        pltpu.sync_copy(srcr_hbm.at[wid, pl.ds(k * ICH, ICH)], srcv)
        pltpu.sync_copy(dstr_hbm.at[wid, pl.ds(k * ICH, ICH)], dstv)

        pltpu.async_copy(g_hbm.at[srcv.at[0]], rows0, sem0)
        pltpu.async_copy(g_hbm.at[srcv.at[1]], rows1, sem1)

        @pl.loop(0, ICH - 2, step=2)
        def _(j):
            pltpu.make_async_copy(g_hbm.at[srcv.at[j]], rows0, sem0).wait()
            pltpu.async_copy(rows0, accum.at[dstv.at[j]], sem2, add=True)
            pltpu.make_async_copy(g_hbm.at[srcv.at[j + 1]], rows1, sem1).wait()
            pltpu.async_copy(rows1, accum.at[dstv.at[j + 1]], sem3, add=True)
            pltpu.make_async_copy(rows0, accum.at[dstv.at[j]], sem2).wait()
            pltpu.async_copy(g_hbm.at[srcv.at[j + 2]], rows0, sem0)
            pltpu.make_async_copy(rows1, accum.at[dstv.at[j + 1]], sem3).wait()
            pltpu.async_copy(g_hbm.at[srcv.at[j + 3]], rows1, sem1)

        pltpu.make_async_copy(g_hbm.at[srcv.at[ICH - 2]], rows0, sem0).wait()
        pltpu.sync_copy(rows0, accum.at[dstv.at[ICH - 2]], add=True)
        pltpu.make_async_copy(g_hbm.at[srcv.at[ICH - 1]], rows1, sem1).wait()
        pltpu.sync_copy(rows1, accum.at[dstv.at[ICH - 1]], add=True)
    plsc.subcore_barrier()

    pltpu.sync_copy(accum.at[pl.ds(s * RPS, RPS)], out_hbm.at[c, pl.ds(s * RPS, RPS)])


_RB = 1280
_GRID = NPAD // _RB

_deg_spec = pl.BlockSpec((NC, _RB), lambda i: (0, i))
_row_spec = pl.BlockSpec((_RB, D), lambda i: (i, 0))
_par_spec = pl.BlockSpec((NC, _RB, D), lambda i: (0, i, 0))


def _norm_of(deg_ref):
    d = (deg_ref[0] + deg_ref[1])[:, None]
    return lax.rsqrt(jnp.maximum(d, 1.0))


def _scale_body(deg_ref, feat_ref, o_ref):
    o_ref[...] = feat_ref[...] * _norm_of(deg_ref)


_scale_call = pl.pallas_call(
    _scale_body,
    grid=(_GRID,),
    in_specs=[_deg_spec, _row_spec],
    out_specs=_row_spec,
    out_shape=jax.ShapeDtypeStruct((NPAD, D), jnp.float32),
)


def _comb_body(deg_ref, p_ref, o_ref):
    d = (deg_ref[0] + deg_ref[1])[:, None]
    o_ref[...] = (p_ref[0] + p_ref[1]) / jnp.maximum(d, 1.0)


_comb_call = pl.pallas_call(
    _comb_body,
    grid=(_GRID,),
    in_specs=[_deg_spec, _par_spec],
    out_specs=_row_spec,
    out_shape=jax.ShapeDtypeStruct((NPAD, D), jnp.float32),
)


def _final_body(deg_ref, p_ref, w_ref, b_ref, o_ref):
    h = (p_ref[0] + p_ref[1]) * _norm_of(deg_ref)
    o_ref[...] = (
        lax.dot_general(h, w_ref[...], (((1,), (1,)), ((), ())),
                        preferred_element_type=jnp.float32)
        + b_ref[...]
    )


_final_call = pl.pallas_call(
    _final_body,
    grid=(_GRID,),
    in_specs=[
        _deg_spec,
        _par_spec,
        pl.BlockSpec((D, D), lambda i: (0, 0)),
        pl.BlockSpec((1, D), lambda i: (0, 0)),
    ],
    out_specs=_row_spec,
    out_shape=jax.ShapeDtypeStruct((NPAD, D), jnp.float32),
)


def kernel(feat, edge_index, W1, b1):
    ppw = EPT - E // NW
    pad_src = jnp.broadcast_to(jnp.arange(ppw, dtype=jnp.int32), (NW, ppw))
    pad_dst = jnp.broadcast_to(N + jnp.arange(ppw, dtype=jnp.int32), (NW, ppw))
    srcp = jnp.concatenate([edge_index[0].reshape(NW, E // NW), pad_src], axis=1)
    srcp = srcp.reshape(NW, NBLK, BLK)
    dstp = jnp.concatenate([edge_index[1].reshape(NW, E // NW), pad_dst], axis=1)
    dstp = dstp.reshape(NW, NBLK, BLK)
    featp = jnp.concatenate([feat, jnp.zeros((NPAD - N, D), feat.dtype)])

    degp = _deg_sc(dstp)
    g1 = _scale_call(degp, featp)
    p1 = _hop_sc(g1, srcp, dstp)
    g2 = _comb_call(degp, p1)
    p2 = _hop_sc(g2, srcp, dstp)
    x = _final_call(degp, p2, W1, b1.reshape(1, D))
    return x[:N]
